# Initial kernel scaffold; baseline (speedup 1.0000x reference)
import jax
import jax.numpy as jnp
from jax.experimental import pallas as pl


def kernel(x, edge_index, batch, W1, b1, W2, b2, Wc1, bc1, Wc2, bc2):
    raise NotImplementedError("write your pallas kernel here")


# speedup vs baseline: 11.9979x; 11.9979x over previous
import functools

import jax
import jax.numpy as jnp
from jax import lax
from jax.experimental import pallas as pl
from jax.experimental.pallas import tpu as pltpu
from jax.experimental.pallas import tpu_sc as plsc

N_NODES = 10000
N_EDGES = 320000
D = 128
NUM_GRAPHS = 64
N_CLASSES = 16

NP = 10240
NB = 10
BLK = 1024
NW = 32
EP = 323584
EPW = EP // NW
CH = 128
NCH = EPW // CH
ROWS_PER_TILE = NP // 16


def _sc_mesh():
    return plsc.VectorSubcoreMesh(core_axis_name="c", subcore_axis_name="s")


def _sc_deg(colp):
    @functools.partial(
        pl.kernel,
        out_type=jax.ShapeDtypeStruct((NW, NP), jnp.float32),
        mesh=_sc_mesh(),
        scratch_types=[
            pltpu.VMEM((EPW,), jnp.int32),
            pltpu.VMEM((NP,), jnp.float32),
        ],
        compiler_params=pltpu.CompilerParams(needs_layout_passes=False),
    )
    def kdeg(col_hbm, out_hbm, colv, degv):
        cid = lax.axis_index("c")
        sid = lax.axis_index("s")
        wid = sid * 2 + cid
        pltpu.sync_copy(col_hbm.at[pl.ds(wid * EPW, EPW)], colv)

        def zero(j, carry):
            degv[pl.ds(j * 16, 16)] = jnp.zeros((16,), jnp.float32)
            return carry

        lax.fori_loop(0, NP // 16, zero, 0)

        ones = jnp.ones((16,), jnp.float32)

        def scat(i, carry):
            idx = colv[pl.ds(i * 16, 16)]
            plsc.addupdate_scatter(degv, [idx], ones)
            return carry

        lax.fori_loop(0, EPW // 16, scat, 0)
        pltpu.sync_copy(degv, out_hbm.at[wid])

    return kdeg(colp)


def _sc_scatter(y, rowp, colp):
    @functools.partial(
        pl.kernel,
        out_type=jax.ShapeDtypeStruct((2, NP, D), jnp.float32),
        mesh=_sc_mesh(),
        scratch_types=[
            pltpu.VMEM((CH,), jnp.int32),
            pltpu.VMEM((CH,), jnp.int32),
            pltpu.VMEM((CH, D), jnp.float32),
            pltpu.VMEM_SHARED((NP, D), jnp.float32),
            pltpu.SemaphoreType.DMA,
        ],
    )
    def kscat(y_hbm, row_hbm, col_hbm, out_hbm, rowv, colv, datav, acc, sem):
        cid = lax.axis_index("c")
        sid = lax.axis_index("s")
        wid = sid * 2 + cid

        def zrow(j, carry):
            for l in range(D // 16):
                datav[j, pl.ds(l * 16, 16)] = jnp.zeros((16,), jnp.float32)
            return carry

        lax.fori_loop(0, CH, zrow, 0)
        for k in range(ROWS_PER_TILE // CH):
            pltpu.sync_copy(
                datav, acc.at[pl.ds(sid * ROWS_PER_TILE + k * CH, CH)]
            )
        plsc.subcore_barrier()

        base = wid * EPW

        def chunk(i, carry):
            off = base + i * CH
            pltpu.sync_copy(row_hbm.at[pl.ds(off, CH)], rowv)
            pltpu.sync_copy(col_hbm.at[pl.ds(off, CH)], colv)
            pltpu.async_copy(y_hbm.at[rowv], datav, sem).wait()
            pltpu.sync_copy(datav, acc.at[colv], add=True)
            return carry

        lax.fori_loop(0, NCH, chunk, 0)
        plsc.subcore_barrier()
        pltpu.sync_copy(
            acc.at[pl.ds(sid * ROWS_PER_TILE, ROWS_PER_TILE)],
            out_hbm.at[cid, pl.ds(sid * ROWS_PER_TILE, ROWS_PER_TILE)],
        )

    return kscat(y, rowp, colp)


def _dinv_of(deg_ref):
    dsum = jnp.sum(deg_ref[...], axis=0)
    return lax.rsqrt(1.0 + dsum)[:, None]


def _y1_body(x_ref, w_ref, deg_ref, y_ref):
    dinv = _dinv_of(deg_ref)
    xw = jnp.dot(x_ref[...], w_ref[...], preferred_element_type=jnp.float32)
    y_ref[...] = xw * dinv


def _tc_y1(xP, W1, degparts):
    return pl.pallas_call(
        _y1_body,
        grid=(NB,),
        in_specs=[
            pl.BlockSpec((BLK, D), lambda i: (i, 0)),
            pl.BlockSpec((D, D), lambda i: (0, 0)),
            pl.BlockSpec((NW, BLK), lambda i: (0, i)),
        ],
        out_specs=pl.BlockSpec((BLK, D), lambda i: (i, 0)),
        out_shape=jax.ShapeDtypeStruct((NP, D), jnp.float32),
    )(xP, W1, degparts)


def _d1_body(p_ref, y_ref, deg_ref, b_ref, w_ref, out_ref):
    dinv = _dinv_of(deg_ref)
    h = jnp.maximum(dinv * (p_ref[0] + p_ref[1] + y_ref[...]) + b_ref[...], 0.0)
    out_ref[...] = dinv * jnp.dot(
        h, w_ref[...], preferred_element_type=jnp.float32
    )


def _tc_d1(parts, y1, degparts, b1r, W2):
    return pl.pallas_call(
        _d1_body,
        grid=(NB,),
        in_specs=[
            pl.BlockSpec((2, BLK, D), lambda i: (0, i, 0)),
            pl.BlockSpec((BLK, D), lambda i: (i, 0)),
            pl.BlockSpec((NW, BLK), lambda i: (0, i)),
            pl.BlockSpec((1, D), lambda i: (0, 0)),
            pl.BlockSpec((D, D), lambda i: (0, 0)),
        ],
        out_specs=pl.BlockSpec((BLK, D), lambda i: (i, 0)),
        out_shape=jax.ShapeDtypeStruct((NP, D), jnp.float32),
    )(parts, y1, degparts, b1r, W2)


def _d2_body(p_ref, y_ref, deg_ref, b_ref, batch_ref, wc1_ref, bc1_ref,
             wc2_ref, bc2_ref, out_ref, psum, cnt):
    i = pl.program_id(0)

    @pl.when(i == 0)
    def _():
        psum[...] = jnp.zeros_like(psum)
        cnt[...] = jnp.zeros_like(cnt)

    dinv = _dinv_of(deg_ref)
    h = jnp.maximum(dinv * (p_ref[0] + p_ref[1] + y_ref[...]) + b_ref[...], 0.0)
    b = batch_ref[0, 0, :]
    oh = (b[:, None] == lax.broadcasted_iota(jnp.int32, (BLK, NUM_GRAPHS), 1))
    oh = oh.astype(jnp.float32)
    psum[...] += lax.dot_general(
        oh, h, (((0,), (0,)), ((), ())), preferred_element_type=jnp.float32
    )
    cnt[...] += jnp.broadcast_to(
        jnp.sum(oh, axis=0)[:, None], (NUM_GRAPHS, D)
    )

    @pl.when(i == NB - 1)
    def _():
        pooled = psum[...] / jnp.maximum(cnt[...], 1.0)
        z = jnp.maximum(
            jnp.dot(pooled, wc1_ref[...], preferred_element_type=jnp.float32)
            + bc1_ref[...],
            0.0,
        )
        out_ref[...] = (
            jnp.dot(z, wc2_ref[...], preferred_element_type=jnp.float32)
            + bc2_ref[...]
        )


def _tc_d2(parts, y2, degparts, b2r, batchP, Wc1, bc1r, Wc2, bc2r):
    return pl.pallas_call(
        _d2_body,
        grid=(NB,),
        in_specs=[
            pl.BlockSpec((2, BLK, D), lambda i: (0, i, 0)),
            pl.BlockSpec((BLK, D), lambda i: (i, 0)),
            pl.BlockSpec((NW, BLK), lambda i: (0, i)),
            pl.BlockSpec((1, D), lambda i: (0, 0)),
            pl.BlockSpec((1, 1, BLK), lambda i: (i, 0, 0)),
            pl.BlockSpec((D, D), lambda i: (0, 0)),
            pl.BlockSpec((1, D), lambda i: (0, 0)),
            pl.BlockSpec((D, N_CLASSES), lambda i: (0, 0)),
            pl.BlockSpec((1, N_CLASSES), lambda i: (0, 0)),
        ],
        out_specs=pl.BlockSpec((NUM_GRAPHS, N_CLASSES), lambda i: (0, 0)),
        out_shape=jax.ShapeDtypeStruct((NUM_GRAPHS, N_CLASSES), jnp.float32),
        scratch_shapes=[
            pltpu.VMEM((NUM_GRAPHS, D), jnp.float32),
            pltpu.VMEM((NUM_GRAPHS, D), jnp.float32),
        ],
    )(parts, y2, degparts, b2r, batchP, Wc1, bc1r, Wc2, bc2r)


def kernel(x, edge_index, batch, W1, b1, W2, b2, Wc1, bc1, Wc2, bc2):
    ei = edge_index.astype(jnp.int32)
    pad_e = EP - N_EDGES
    rowp = jnp.concatenate([ei[0], jnp.zeros((pad_e,), jnp.int32)])
    colp = jnp.concatenate(
        [ei[1], jnp.full((pad_e,), N_NODES, jnp.int32)]
    )
    xP = jnp.concatenate(
        [x, jnp.zeros((NP - N_NODES, D), jnp.float32)]
    )
    batchP = jnp.concatenate(
        [batch.astype(jnp.int32),
         jnp.full((NP - N_NODES,), NUM_GRAPHS, jnp.int32)]
    ).reshape(NB, 1, BLK)
    b1r = b1.reshape(1, D)
    b2r = b2.reshape(1, D)
    bc1r = bc1.reshape(1, D)
    bc2r = bc2.reshape(1, N_CLASSES)

    degparts = _sc_deg(colp)
    y1 = _tc_y1(xP, W1, degparts)
    p1 = _sc_scatter(y1, rowp, colp)
    y2 = _tc_d1(p1, y1, degparts, b1r, W2)
    p2 = _sc_scatter(y2, rowp, colp)
    return _tc_d2(p2, y2, degparts, b2r, batchP, Wc1, bc1r, Wc2, bc2r)


# speedup vs baseline: 14.0951x; 1.1748x over previous
import functools

import jax
import jax.numpy as jnp
from jax import lax
from jax.experimental import pallas as pl
from jax.experimental.pallas import tpu as pltpu
from jax.experimental.pallas import tpu_sc as plsc

N_NODES = 10000
N_EDGES = 320000
D = 128
DH = D // 2
NUM_GRAPHS = 64
N_CLASSES = 16

NP = 10240
NB = 10
BLK = 1024
EP = 327680
CH = 128
NCHT = EP // (16 * CH)
RING = 4
ROWS_PER_TILE = NP // 16


def _sc_mesh():
    return plsc.VectorSubcoreMesh(core_axis_name="c", subcore_axis_name="s")


def _sc_deg(colp):
    EPW = EP // 32

    @functools.partial(
        pl.kernel,
        out_type=jax.ShapeDtypeStruct((32, NP), jnp.float32),
        mesh=_sc_mesh(),
        scratch_types=[
            pltpu.VMEM((EPW,), jnp.int32),
            pltpu.VMEM((NP,), jnp.float32),
        ],
        compiler_params=pltpu.CompilerParams(needs_layout_passes=False),
    )
    def kdeg(col_hbm, out_hbm, colv, degv):
        cid = lax.axis_index("c")
        sid = lax.axis_index("s")
        wid = sid * 2 + cid
        pltpu.sync_copy(col_hbm.at[pl.ds(wid * EPW, EPW)], colv)

        def zero(j, carry):
            degv[pl.ds(j * 16, 16)] = jnp.zeros((16,), jnp.float32)
            return carry

        lax.fori_loop(0, NP // 16, zero, 0)

        ones = jnp.ones((16,), jnp.float32)

        def scat(i, carry):
            idx = colv[pl.ds(i * 16, 16)]
            plsc.addupdate_scatter(degv, [idx], ones)
            return carry

        lax.fori_loop(0, EPW // 16, scat, 0)
        pltpu.sync_copy(degv, out_hbm.at[wid])

    return kdeg(colp)


def _sc_scatter(ys, rowp2, colp2):
    @functools.partial(
        pl.kernel,
        out_type=jax.ShapeDtypeStruct((2, NP, DH), jnp.float32),
        mesh=_sc_mesh(),
        scratch_types=[
            pltpu.VMEM((NCHT, CH), jnp.int32),
            pltpu.VMEM((NCHT, CH), jnp.int32),
            pltpu.VMEM((RING, CH, DH), jnp.float32),
            pltpu.VMEM_SHARED((NP, DH), jnp.float32),
        ]
        + [pltpu.SemaphoreType.DMA] * (2 * RING),
        compiler_params=pltpu.CompilerParams(use_tc_tiling_on_sc=False),
    )
    def kscat(y_hbm, row_hbm, col_hbm, out_hbm, rowv, colv, datav, acc, *sems):
        gsems = sems[:RING]
        ssems = sems[RING:]
        cid = lax.axis_index("c")
        sid = lax.axis_index("s")
        ysrc = y_hbm.at[cid]

        def zrow(j, carry):
            for l in range(DH // 16):
                datav[0, j, pl.ds(l * 16, 16)] = jnp.zeros((16,), jnp.float32)
            return carry

        lax.fori_loop(0, CH, zrow, 0)
        for k in range(ROWS_PER_TILE // CH):
            pltpu.sync_copy(
                datav.at[0], acc.at[pl.ds(sid * ROWS_PER_TILE + k * CH, CH)]
            )
        plsc.subcore_barrier()

        pltpu.sync_copy(row_hbm.at[pl.ds(sid * NCHT, NCHT)], rowv)
        pltpu.sync_copy(col_hbm.at[pl.ds(sid * NCHT, NCHT)], colv)

        for b in range(RING):
            pltpu.async_copy(ysrc.at[rowv.at[b]], datav.at[b], gsems[b])

        def outer(o, carry):
            for b in range(RING):
                g = o * RING + b
                pltpu.make_async_copy(
                    ysrc.at[rowv.at[g]], datav.at[b], gsems[b]
                ).wait()
                pltpu.async_copy(
                    datav.at[b], acc.at[colv.at[g]], ssems[b], add=True
                ).wait()
                pltpu.async_copy(
                    ysrc.at[rowv.at[g + RING]], datav.at[b], gsems[b]
                )
            return carry

        lax.fori_loop(0, NCHT // RING - 1, outer, 0)
        for b in range(RING):
            g = NCHT - RING + b
            pltpu.make_async_copy(
                ysrc.at[rowv.at[g]], datav.at[b], gsems[b]
            ).wait()
            pltpu.async_copy(
                datav.at[b], acc.at[colv.at[g]], ssems[b], add=True
            ).wait()

        plsc.subcore_barrier()
        pltpu.sync_copy(
            acc.at[pl.ds(sid * ROWS_PER_TILE, ROWS_PER_TILE)],
            out_hbm.at[cid, pl.ds(sid * ROWS_PER_TILE, ROWS_PER_TILE)],
        )

    return kscat(ys, rowp2, colp2)


def _dinv_of(deg_ref):
    dsum = jnp.sum(deg_ref[...], axis=0)
    return lax.rsqrt(1.0 + dsum)[:, None]


def _split(ref):
    return jnp.concatenate([ref[0], ref[1]], axis=1)


def _store_split(ref, val):
    ref[0] = val[:, :DH]
    ref[1] = val[:, DH:]


def _y1_body(x_ref, w_ref, deg_ref, y_ref):
    dinv = _dinv_of(deg_ref)
    xw = jnp.dot(x_ref[...], w_ref[...], preferred_element_type=jnp.float32)
    _store_split(y_ref, xw * dinv)


def _tc_y1(xP, W1, degparts):
    return pl.pallas_call(
        _y1_body,
        grid=(NB,),
        in_specs=[
            pl.BlockSpec((BLK, D), lambda i: (i, 0)),
            pl.BlockSpec((D, D), lambda i: (0, 0)),
            pl.BlockSpec((32, BLK), lambda i: (0, i)),
        ],
        out_specs=pl.BlockSpec((2, BLK, DH), lambda i: (0, i, 0)),
        out_shape=jax.ShapeDtypeStruct((2, NP, DH), jnp.float32),
    )(xP, W1, degparts)


def _d1_body(p_ref, y_ref, deg_ref, b_ref, w_ref, out_ref):
    dinv = _dinv_of(deg_ref)
    h = jnp.maximum(
        dinv * (_split(p_ref) + _split(y_ref)) + b_ref[...], 0.0
    )
    xw = jnp.dot(h, w_ref[...], preferred_element_type=jnp.float32)
    _store_split(out_ref, xw * dinv)


def _tc_d1(parts, y1, degparts, b1r, W2):
    return pl.pallas_call(
        _d1_body,
        grid=(NB,),
        in_specs=[
            pl.BlockSpec((2, BLK, DH), lambda i: (0, i, 0)),
            pl.BlockSpec((2, BLK, DH), lambda i: (0, i, 0)),
            pl.BlockSpec((32, BLK), lambda i: (0, i)),
            pl.BlockSpec((1, D), lambda i: (0, 0)),
            pl.BlockSpec((D, D), lambda i: (0, 0)),
        ],
        out_specs=pl.BlockSpec((2, BLK, DH), lambda i: (0, i, 0)),
        out_shape=jax.ShapeDtypeStruct((2, NP, DH), jnp.float32),
    )(parts, y1, degparts, b1r, W2)


def _d2_body(p_ref, y_ref, deg_ref, b_ref, batch_ref, wc1_ref, bc1_ref,
             wc2_ref, bc2_ref, out_ref, psum, cnt):
    i = pl.program_id(0)

    @pl.when(i == 0)
    def _():
        psum[...] = jnp.zeros_like(psum)
        cnt[...] = jnp.zeros_like(cnt)

    dinv = _dinv_of(deg_ref)
    h = jnp.maximum(
        dinv * (_split(p_ref) + _split(y_ref)) + b_ref[...], 0.0
    )
    b = batch_ref[0, 0, :]
    oh = (b[:, None] == lax.broadcasted_iota(jnp.int32, (BLK, NUM_GRAPHS), 1))
    oh = oh.astype(jnp.float32)
    psum[...] += lax.dot_general(
        oh, h, (((0,), (0,)), ((), ())), preferred_element_type=jnp.float32
    )
    cnt[...] += jnp.broadcast_to(
        jnp.sum(oh, axis=0)[:, None], (NUM_GRAPHS, D)
    )

    @pl.when(i == NB - 1)
    def _():
        pooled = psum[...] / jnp.maximum(cnt[...], 1.0)
        z = jnp.maximum(
            jnp.dot(pooled, wc1_ref[...], preferred_element_type=jnp.float32)
            + bc1_ref[...],
            0.0,
        )
        out_ref[...] = (
            jnp.dot(z, wc2_ref[...], preferred_element_type=jnp.float32)
            + bc2_ref[...]
        )


def _tc_d2(parts, y2, degparts, b2r, batchP, Wc1, bc1r, Wc2, bc2r):
    return pl.pallas_call(
        _d2_body,
        grid=(NB,),
        in_specs=[
            pl.BlockSpec((2, BLK, DH), lambda i: (0, i, 0)),
            pl.BlockSpec((2, BLK, DH), lambda i: (0, i, 0)),
            pl.BlockSpec((32, BLK), lambda i: (0, i)),
            pl.BlockSpec((1, D), lambda i: (0, 0)),
            pl.BlockSpec((1, 1, BLK), lambda i: (i, 0, 0)),
            pl.BlockSpec((D, D), lambda i: (0, 0)),
            pl.BlockSpec((1, D), lambda i: (0, 0)),
            pl.BlockSpec((D, N_CLASSES), lambda i: (0, 0)),
            pl.BlockSpec((1, N_CLASSES), lambda i: (0, 0)),
        ],
        out_specs=pl.BlockSpec((NUM_GRAPHS, N_CLASSES), lambda i: (0, 0)),
        out_shape=jax.ShapeDtypeStruct((NUM_GRAPHS, N_CLASSES), jnp.float32),
        scratch_shapes=[
            pltpu.VMEM((NUM_GRAPHS, D), jnp.float32),
            pltpu.VMEM((NUM_GRAPHS, D), jnp.float32),
        ],
    )(parts, y2, degparts, b2r, batchP, Wc1, bc1r, Wc2, bc2r)


def kernel(x, edge_index, batch, W1, b1, W2, b2, Wc1, bc1, Wc2, bc2):
    ei = edge_index.astype(jnp.int32)
    pad_e = EP - N_EDGES
    rowp = jnp.concatenate([ei[0], jnp.zeros((pad_e,), jnp.int32)])
    colp = jnp.concatenate(
        [ei[1], jnp.full((pad_e,), N_NODES, jnp.int32)]
    )
    rowp2 = rowp.reshape(EP // CH, CH)
    colp2 = colp.reshape(EP // CH, CH)
    xP = jnp.concatenate(
        [x, jnp.zeros((NP - N_NODES, D), jnp.float32)]
    )
    batchP = jnp.concatenate(
        [batch.astype(jnp.int32),
         jnp.full((NP - N_NODES,), NUM_GRAPHS, jnp.int32)]
    ).reshape(NB, 1, BLK)
    b1r = b1.reshape(1, D)
    b2r = b2.reshape(1, D)
    bc1r = bc1.reshape(1, D)
    bc2r = bc2.reshape(1, N_CLASSES)

    degparts = _sc_deg(colp)
    y1 = _tc_y1(xP, W1, degparts)
    p1 = _sc_scatter(y1, rowp2, colp2)
    y2 = _tc_d1(p1, y1, degparts, b1r, W2)
    p2 = _sc_scatter(y2, rowp2, colp2)
    return _tc_d2(p2, y2, degparts, b2r, batchP, Wc1, bc1r, Wc2, bc2r)
